# baseline (speedup 1.0000x reference)
import functools

import jax
import jax.numpy as jnp
from jax import lax
from jax.experimental import pallas as pl
from jax.experimental.pallas import tpu as pltpu
from jax.experimental.pallas import tpu_sc as plsc

_LANES = 16


@functools.lru_cache(maxsize=None)
def _build(B, S, E, V):
    info = plsc.get_sparse_core_info()
    nw = info.num_cores * info.num_subcores
    assert B % nw == 0, (B, nw)
    assert E % _LANES == 0 and S % 2 == 0
    rpw = B // nw
    assert rpw >= 6 and rpw % 2 == 0
    e_vecs = E // _LANES
    s2 = S // 2
    wide = 2 * E
    chunks = []
    off = 0
    while off < S:
        sz = min(128, S - off)
        chunks.append((off, sz))
        off += sz

    mesh = plsc.VectorSubcoreMesh(core_axis_name="c", subcore_axis_name="s")

    @functools.partial(
        pl.kernel,
        mesh=mesh,
        out_type=jax.ShapeDtypeStruct((B, s2, wide), jnp.float32),
        scratch_types=[
            pltpu.VMEM((rpw * S,), jnp.int32),
            pltpu.VMEM((3, S, E), jnp.float32),
            pltpu.VMEM((3, s2, wide), jnp.float32),
            pltpu.VMEM((s2, wide), jnp.float32),
            pltpu.SemaphoreType.DMA,
            pltpu.SemaphoreType.DMA,
            pltpu.SemaphoreType.DMA,
            pltpu.SemaphoreType.DMA,
            pltpu.SemaphoreType.DMA,
            pltpu.SemaphoreType.DMA,
        ],
        compiler_params=pltpu.CompilerParams(use_tc_tiling_on_sc=False),
    )
    def k(x_hbm, tok_hbm, pos_hbm, out_hbm, idx_v, g_v, rows_v, pos_v,
          sg0, sg1, sg2, ss0, ss1, ss2):
        wid = lax.axis_index("s") * info.num_cores + lax.axis_index("c")
        base = wid * rpw
        sem_g = (sg0, sg1, sg2)
        sem_s = (ss0, ss1, ss2)

        pltpu.sync_copy(x_hbm.at[pl.ds(base * S, rpw * S)], idx_v)
        pltpu.sync_copy(pos_hbm, pos_v)

        def fetch(i, u):
            for off, sz in chunks:
                pltpu.async_copy(
                    tok_hbm.at[idx_v.at[pl.ds(i * S + off, sz)]],
                    g_v.at[u].at[pl.ds(off, sz)],
                    sem_g[u])

        def wait_g(u):
            pltpu.make_async_copy(
                tok_hbm.at[pl.ds(0, S)], g_v.at[u], sem_g[u]).wait()

        def store(i, u):
            pltpu.async_copy(rows_v.at[u], out_hbm.at[base + i], sem_s[u])

        def wait_s(u):
            pltpu.make_async_copy(out_hbm.at[0], rows_v.at[u], sem_s[u]).wait()

        assert s2 % 2 == 0

        def add_pos(u):
            def body(q, _):
                for dp in (0, 1):
                    p = 2 * q + dp
                    for h in (0, 1):
                        for j in range(e_vecs):
                            src = pl.ds(j * _LANES, _LANES)
                            dst = pl.ds(h * E + j * _LANES, _LANES)
                            rows_v[u, p, dst] = (
                                g_v[u, 2 * p + h, src] + pos_v[p, dst])
                return 0
            lax.fori_loop(0, s2 // 2, body, 0)

        assert rpw % 3 == 2 and rpw >= 8
        fetch(0, 0)
        fetch(1, 1)

        def iteration(i, u, pre, w_s):
            wait_g(u)
            if pre:
                fetch(i + 2, (u + 2) % 3)
            if w_s:
                wait_s(u)
            add_pos(u)
            store(i, u)

        iteration(0, 0, True, False)
        iteration(1, 1, True, False)
        iteration(2, 2, True, False)

        def group(g, _):
            for uu in (0, 1, 2):
                iteration(3 + 3 * g + uu, uu, True, True)
            return 0

        lax.fori_loop(0, (rpw - 5) // 3, group, 0)

        iteration(rpw - 2, (rpw - 2) % 3, False, True)
        iteration(rpw - 1, (rpw - 1) % 3, False, True)

        wait_s((rpw - 3) % 3)
        wait_s((rpw - 2) % 3)
        wait_s((rpw - 1) % 3)

    return k


def kernel(x, token_table, pos_table):
    B, S = x.shape
    V, E = token_table.shape
    k = _build(B, S, E, V)
    pos2 = pos_table.reshape(S // 2, 2 * E)
    x1 = x.astype(jnp.int32).reshape(B * S)
    out = k(x1, token_table, pos2)
    return out.reshape(B, S, E)
